# baseline (speedup 1.0000x reference)
import functools

import jax
import jax.numpy as jnp
from jax import lax
from jax.experimental import pallas as pl
from jax.experimental.pallas import tpu as pltpu
from jax.experimental.pallas import tpu_sc as plsc

NUM_CLASS = 1000
ALPHA = 0.25
GAMMA = 2.0
EPS = 1e-10

ROWS = 32768
PAIRS = ROWS // 2
N_CHUNKS = 4
CHUNK_R = PAIRS // N_CHUNKS
BLOCK_R = 2048
BLOCKS_PER_CHUNK = CHUNK_R // BLOCK_R

NC = 2
NS = 16
NW = NC * NS
P_PER_W = PAIRS // NW
CH = 128
N_CH = P_PER_W // CH


def _sim_sc_kernel(desc_hbm, out_hbm, a_v, p_v, o_v):
    wid = lax.axis_index("s") * NC + lax.axis_index("c")
    base = wid * P_PER_W

    @pl.loop(0, N_CH)
    def _chunk(ci):
        row0 = base + ci * CH
        pltpu.sync_copy(desc_hbm.at[pl.ds(row0, CH)], a_v)
        pltpu.sync_copy(desc_hbm.at[pl.ds(PAIRS + row0, CH)], p_v)
        for r in range(CH):
            acc = a_v[r, pl.ds(0, 16)] * p_v[r, pl.ds(0, 16)]
            for k in range(1, 8):
                acc = acc + a_v[r, pl.ds(16 * k, 16)] * p_v[r, pl.ds(16 * k, 16)]
            o_v[r, :] = acc
        pltpu.sync_copy(o_v, out_hbm.at[pl.ds(row0, CH)])


@functools.cache
def _sim_sc():
    return pl.kernel(
        _sim_sc_kernel,
        out_type=jax.ShapeDtypeStruct((PAIRS, 16), jnp.float32),
        mesh=plsc.VectorSubcoreMesh(
            core_axis_name="c", subcore_axis_name="s",
            num_cores=NC, num_subcores=NS),
        scratch_types=[
            pltpu.VMEM((CH, 128), jnp.float32),
            pltpu.VMEM((CH, 128), jnp.float32),
            pltpu.VMEM((CH, 16), jnp.float32),
        ],
    )


def _logpt(x, t):
    row_max = jnp.max(x, axis=1, keepdims=True)
    sumexp = jnp.sum(jnp.exp(x - row_max), axis=1, keepdims=True)
    cols = jax.lax.broadcasted_iota(jnp.int32, x.shape, 1)
    xt = jnp.sum(jnp.where(cols == t, x, 0.0), axis=1, keepdims=True)
    pt = jnp.exp(xt - row_max) / sumexp
    return jnp.log(pt + EPS)


def _lp_kernel(xlo_ref, xhi_ref, tlo_ref, thi_ref, out_ref):
    out_ref[...] = (_logpt(xlo_ref[...], tlo_ref[...])
                    + _logpt(xhi_ref[...], thi_ref[...]))


def _combine_kernel(sim_ref, lp0_ref, lp1_ref, lp2_ref, lp3_ref, out_ref):
    sim = jnp.sum(sim_ref[...], axis=1, keepdims=True)
    omp = 1.0 - sim
    weight = -ALPHA * omp * omp
    lp = jnp.concatenate(
        [lp0_ref[...], lp1_ref[...], lp2_ref[...], lp3_ref[...]], axis=0)
    out_ref[...] = jnp.sum(weight * lp).reshape(1, 1)


def _lp_chunk(xlo, xhi, tlo, thi):
    return pl.pallas_call(
        _lp_kernel,
        grid=(BLOCKS_PER_CHUNK,),
        in_specs=[
            pl.BlockSpec((BLOCK_R, NUM_CLASS), lambda i: (i, 0)),
            pl.BlockSpec((BLOCK_R, NUM_CLASS), lambda i: (i, 0)),
            pl.BlockSpec((BLOCK_R, 1), lambda i: (i, 0)),
            pl.BlockSpec((BLOCK_R, 1), lambda i: (i, 0)),
        ],
        out_specs=pl.BlockSpec((BLOCK_R, 1), lambda i: (i, 0)),
        out_shape=jax.ShapeDtypeStruct((CHUNK_R, 1), jnp.float32),
        compiler_params=pltpu.CompilerParams(
            dimension_semantics=("parallel",)),
    )(xlo, xhi, tlo, thi)


@jax.jit
def kernel(descriptors, input, target):
    sim16 = _sim_sc()(descriptors)

    tgt2d = target.reshape(ROWS, 1)
    lps = []
    for c in range(N_CHUNKS):
        r0 = c * CHUNK_R
        xlo = lax.slice(input, (r0, 0), (r0 + CHUNK_R, NUM_CLASS))
        xhi = lax.slice(input, (PAIRS + r0, 0),
                        (PAIRS + r0 + CHUNK_R, NUM_CLASS))
        tlo = lax.slice(tgt2d, (r0, 0), (r0 + CHUNK_R, 1))
        thi = lax.slice(tgt2d, (PAIRS + r0, 0), (PAIRS + r0 + CHUNK_R, 1))
        lps.append(_lp_chunk(xlo, xhi, tlo, thi))

    total = pl.pallas_call(
        _combine_kernel,
        out_shape=jax.ShapeDtypeStruct((1, 1), jnp.float32),
    )(sim16, *lps)
    return total[0, 0] / ROWS


# speedup vs baseline: 1.2904x; 1.2904x over previous
import jax
import jax.numpy as jnp
from jax.experimental import pallas as pl
from jax.experimental.pallas import tpu as pltpu

NUM_CLASS = 1000
PAD_CLASS = 1024
ALPHA = 0.25
GAMMA = 2.0
EPS = 1e-10
NEG_BIG = -1e30

ROWS = 32768
PAIRS = ROWS // 2
BLOCK_R = 2048
N_BLOCKS = PAIRS // BLOCK_R


def _logpt(x, t):
    row_max = jnp.max(x, axis=1, keepdims=True)
    sumexp = jnp.sum(jnp.exp(x - row_max), axis=1, keepdims=True)
    cols = jax.lax.broadcasted_iota(jnp.int32, x.shape, 1)
    xt = jnp.sum(jnp.where(cols == t, x, 0.0), axis=1, keepdims=True)
    pt = jnp.exp(xt - row_max) / sumexp
    return jnp.log(pt + EPS)


def _loss_kernel(xlo_ref, xhi_ref, tlo_ref, thi_ref, anc_ref, pos_ref,
                 out_ref):
    sim = jnp.sum(anc_ref[...] * pos_ref[...], axis=1, keepdims=True)
    omp = 1.0 - sim
    weight = -ALPHA * omp * omp
    lp = _logpt(xlo_ref[...], tlo_ref[...]) + _logpt(xhi_ref[...], thi_ref[...])
    out_ref[...] = jnp.sum(weight * lp).reshape(1, 1, 1)


@jax.jit
def kernel(descriptors, input, target):
    xpad = jnp.pad(input, ((0, 0), (0, PAD_CLASS - NUM_CLASS)),
                   constant_values=NEG_BIG)
    tgt2d = target.reshape(ROWS, 1)
    partials = pl.pallas_call(
        _loss_kernel,
        grid=(N_BLOCKS,),
        in_specs=[
            pl.BlockSpec((BLOCK_R, PAD_CLASS), lambda i: (i, 0)),
            pl.BlockSpec((BLOCK_R, PAD_CLASS), lambda i: (i + N_BLOCKS, 0)),
            pl.BlockSpec((BLOCK_R, 1), lambda i: (i, 0)),
            pl.BlockSpec((BLOCK_R, 1), lambda i: (i + N_BLOCKS, 0)),
            pl.BlockSpec((BLOCK_R, 128), lambda i: (i, 0)),
            pl.BlockSpec((BLOCK_R, 128), lambda i: (i + N_BLOCKS, 0)),
        ],
        out_specs=pl.BlockSpec((1, 1, 1), lambda i: (i, 0, 0)),
        out_shape=jax.ShapeDtypeStruct((N_BLOCKS, 1, 1), jnp.float32),
        compiler_params=pltpu.CompilerParams(
            dimension_semantics=("parallel",)),
    )(xpad, xpad, tgt2d, tgt2d, descriptors, descriptors)
    return jnp.sum(partials) / ROWS


# speedup vs baseline: 5.9890x; 4.6411x over previous
import jax
import jax.numpy as jnp
from jax.experimental import pallas as pl
from jax.experimental.pallas import tpu as pltpu

NUM_CLASS = 1000
ALPHA = 0.25
GAMMA = 2.0
EPS = 1e-10

ROWS = 32768
PAIRS = ROWS // 2
BLOCK_S = 2048
N_BLOCKS = PAIRS // BLOCK_S


def _logpt(x, t):
    col_max = jnp.max(x, axis=0, keepdims=True)
    sumexp = jnp.sum(jnp.exp(x - col_max), axis=0, keepdims=True)
    rows = jax.lax.broadcasted_iota(jnp.int32, x.shape, 0)
    xt = jnp.sum(jnp.where(rows == t, x, 0.0), axis=0, keepdims=True)
    pt = jnp.exp(xt - col_max) / sumexp
    return jnp.log(pt + EPS)


def _loss_kernel(xlo_ref, xhi_ref, tlo_ref, thi_ref, anc_ref, pos_ref,
                 out_ref):
    sim = jnp.sum(anc_ref[...] * pos_ref[...], axis=1, keepdims=True)
    omp = 1.0 - sim
    weight = -ALPHA * omp * omp
    lp = _logpt(xlo_ref[...], tlo_ref[...]) + _logpt(xhi_ref[...], thi_ref[...])
    out_ref[...] = jnp.dot(lp, weight,
                           preferred_element_type=jnp.float32).reshape(1, 1, 1)


@jax.jit
def kernel(descriptors, input, target):
    xt_view = input.T
    tgt2d = target.reshape(1, ROWS)
    partials = pl.pallas_call(
        _loss_kernel,
        grid=(N_BLOCKS,),
        in_specs=[
            pl.BlockSpec((NUM_CLASS, BLOCK_S), lambda i: (0, i)),
            pl.BlockSpec((NUM_CLASS, BLOCK_S), lambda i: (0, i + N_BLOCKS)),
            pl.BlockSpec((1, BLOCK_S), lambda i: (0, i)),
            pl.BlockSpec((1, BLOCK_S), lambda i: (0, i + N_BLOCKS)),
            pl.BlockSpec((BLOCK_S, 128), lambda i: (i, 0)),
            pl.BlockSpec((BLOCK_S, 128), lambda i: (i + N_BLOCKS, 0)),
        ],
        out_specs=pl.BlockSpec((1, 1, 1), lambda i: (i, 0, 0)),
        out_shape=jax.ShapeDtypeStruct((N_BLOCKS, 1, 1), jnp.float32),
        compiler_params=pltpu.CompilerParams(
            dimension_semantics=("parallel",)),
    )(xt_view, xt_view, tgt2d, tgt2d, descriptors, descriptors)
    return jnp.sum(partials) / ROWS
